# baseline (speedup 1.0000x reference)
import jax
import jax.numpy as jnp
from jax.experimental import pallas as pl
from jax.experimental.pallas import tpu as pltpu


def _copy_block(x_ref, o_ref):
    o_ref[...] = x_ref[...]


def _copy_2d(x):
    b, t = x.shape
    rblk = 32
    if b % rblk != 0:
        rblk = b
    grid = b // rblk
    return pl.pallas_call(
        _copy_block,
        out_shape=jax.ShapeDtypeStruct((b, t), x.dtype),
        grid=(grid,),
        in_specs=[pl.BlockSpec((rblk, t), lambda i: (i, 0))],
        out_specs=pl.BlockSpec((rblk, t), lambda i: (i, 0)),
    )(x)


def kernel(x):
    squeeze_batch = False
    if x.ndim == 1:
        x = x[None, :]
        squeeze_batch = True
    out = _copy_2d(x)
    if squeeze_batch:
        out = out[0]
    return out


# speedup vs baseline: 1.1682x; 1.1682x over previous
import jax
import jax.numpy as jnp
from jax.experimental import pallas as pl
from jax.experimental.pallas import tpu as pltpu


def _copy_block(x_ref, o_ref):
    o_ref[...] = x_ref[...]


def _copy_2d(x):
    b, t = x.shape
    rblk = 16
    if b % rblk != 0:
        rblk = b
    grid = b // rblk
    return pl.pallas_call(
        _copy_block,
        out_shape=jax.ShapeDtypeStruct((b, t), x.dtype),
        grid=(grid,),
        in_specs=[pl.BlockSpec((rblk, t), lambda i: (i, 0))],
        out_specs=pl.BlockSpec((rblk, t), lambda i: (i, 0)),
    )(x)


def kernel(x):
    squeeze_batch = False
    if x.ndim == 1:
        x = x[None, :]
        squeeze_batch = True
    out = _copy_2d(x)
    if squeeze_batch:
        out = out[0]
    return out
